# baseline (speedup 1.0000x reference)
import functools

import jax
import jax.numpy as jnp
from jax import lax
from jax.experimental import pallas as pl
from jax.experimental.pallas import tpu as pltpu
from jax.experimental.pallas import tpu_sc as plsc

N = 50000
E = 800000
G = 32
F = 64
H = 32

NP = 51200
EP = 819200
EPG = 128
GPC = 2
CHUNK = EPG * GPC
NSUB = 16
RPS = NP // NSUB
BN = 2048
NB = NP // BN

_MESH = plsc.VectorSubcoreMesh(core_axis_name="c", subcore_axis_name="s")
_f32 = jnp.float32
_i32 = jnp.int32


def _zero_slice(zsrc, acc, s):
    pltpu.sync_copy(zsrc, acc.at[pl.ds(s * RPS, RPS)])


CH1 = 3200


def _zero_local(accl):
    def zb(i, cc):
        accl[pl.ds(i * 16, 16)] = jnp.zeros((16,), _f32)
        return cc
    lax.fori_loop(0, NP // 16, zb, jnp.int32(0))


def _w1_writeback(accl, out_hbm, wid):
    pltpu.sync_copy(accl, out_hbm.at[wid])


def _sc_deg_body(col_hbm, w_hbm, out_hbm, cbufv, wbuf, accl):
    c = lax.axis_index("c")
    s = lax.axis_index("s")
    wid = c * NSUB + s
    _zero_local(accl)

    def chunk(ci, carry):
        base = wid * 25600 + ci * CH1
        pltpu.sync_copy(col_hbm.at[pl.ds(base, CH1)], cbufv)
        pltpu.sync_copy(w_hbm.at[pl.ds(base, CH1)], wbuf)

        def blk(i, cc):
            cv = cbufv[pl.ds(i * 16, 16)]
            wv = wbuf[pl.ds(i * 16, 16)]
            plsc.addupdate_scatter(accl, [cv], wv)
            return cc

        lax.fori_loop(0, CH1 // 16, blk, jnp.int32(0))
        return carry

    lax.fori_loop(0, 25600 // CH1, chunk, jnp.int32(0))
    _w1_writeback(accl, out_hbm, wid)


_sc_deg = functools.partial(
    pl.kernel,
    out_type=jax.ShapeDtypeStruct((32, NP), _f32),
    mesh=_MESH,
    compiler_params=pltpu.CompilerParams(needs_layout_passes=False, use_tc_tiling_on_sc=False),
    scratch_types=[
        pltpu.VMEM((CH1,), _i32),
        pltpu.VMEM((CH1,), _f32),
        pltpu.VMEM((NP,), _f32),
    ],
)(_sc_deg_body)


def _hop32_gather(m_hbm, ebuf, gbuf, sem, gbase):
    return [
        pltpu.async_copy(
            m_hbm.at[ebuf.at[g, 0]], gbuf.at[pl.ds(g * EPG, EPG)], sem
        )
        for g in range(GPC)
    ]


def _hop32_drain_gather(m_hbm, ebuf, gbuf, sem):
    for g in range(GPC):
        pltpu.make_async_copy(
            m_hbm.at[ebuf.at[g, 0]], gbuf.at[pl.ds(g * EPG, EPG)], sem
        ).wait()


def _hop32_scale(ebuf, gbuf):
    for g in range(GPC):
        def blk(i, cc, g=g):
            wv = plsc.bitcast(ebuf[g, 2, pl.ds(i * 16, 16)], _f32)
            for jj in range(16):
                e = g * EPG + i * 16 + jj
                sp = lax.gather(
                    wv, jnp.full((16, 1), jj, _i32),
                    lax.GatherDimensionNumbers(
                        offset_dims=(), collapsed_slice_dims=(0,),
                        start_index_map=(0,)),
                    (1,), mode=lax.GatherScatterMode.PROMISE_IN_BOUNDS)
                gbuf[e, pl.ds(0, 16)] = gbuf[e, pl.ds(0, 16)] * sp
                gbuf[e, pl.ds(16, 16)] = gbuf[e, pl.ds(16, 16)] * sp
            return cc
        lax.fori_loop(0, EPG // 16, blk, jnp.int32(0))


def _hop32_scatter(acc, ebuf, gbuf, sem):
    return [
        pltpu.async_copy(
            gbuf.at[pl.ds(g * EPG, EPG)], acc.at[ebuf.at[g, 1]], sem, add=True
        )
        for g in range(GPC)
    ]


def _hop32_half(m_hbm, t_hbm, s, e_hbm, z2_hbm, ebuf0, ebuf1, gbuf0, gbuf1,
                acc, gsem0, gsem1, ssem0, ssem1, isem0, isem1):
    pltpu.sync_copy(z2_hbm, acc.at[pl.ds(s * RPS, RPS)])
    plsc.subcore_barrier()
    nch = 400 // GPC
    gps = s * 400

    pltpu.sync_copy(e_hbm.at[pl.ds(gps, GPC)], ebuf0)
    pltpu.sync_copy(e_hbm.at[pl.ds(gps + GPC, GPC)], ebuf1)
    _hop32_gather(m_hbm, ebuf0, gbuf0, gsem0, 0)
    _hop32_gather(m_hbm, ebuf1, gbuf1, gsem1, 0)

    def body(i, carry):
        p0 = jnp.minimum(2 * i + 2, nch - 1)
        p1 = jnp.minimum(2 * i + 3, nch - 1)
        _hop32_drain_gather(m_hbm, ebuf0, gbuf0, gsem0)
        _hop32_scale(ebuf0, gbuf0)
        s0 = _hop32_scatter(acc, ebuf0, gbuf0, ssem0)
        _hop32_drain_gather(m_hbm, ebuf1, gbuf1, gsem1)
        _hop32_scale(ebuf1, gbuf1)
        s1 = _hop32_scatter(acc, ebuf1, gbuf1, ssem1)
        for d in s0:
            d.wait()
        i0 = pltpu.async_copy(e_hbm.at[pl.ds(gps + p0 * GPC, GPC)], ebuf0,
                              isem0)
        for d in s1:
            d.wait()
        i1 = pltpu.async_copy(e_hbm.at[pl.ds(gps + p1 * GPC, GPC)], ebuf1,
                              isem1)
        i0.wait()
        _hop32_gather(m_hbm, ebuf0, gbuf0, gsem0, 0)
        i1.wait()
        _hop32_gather(m_hbm, ebuf1, gbuf1, gsem1, 0)
        return carry

    lax.fori_loop(0, nch // 2, body, jnp.int32(0))
    _hop32_drain_gather(m_hbm, ebuf0, gbuf0, gsem0)
    _hop32_drain_gather(m_hbm, ebuf1, gbuf1, gsem1)
    plsc.subcore_barrier()
    pltpu.sync_copy(acc.at[pl.ds(s * RPS, RPS)], t_hbm.at[pl.ds(s * RPS, RPS)])


def _sc_hop32_body(ma_hbm, mb_hbm, e_hbm, z2_hbm, ta_hbm, tb_hbm, ebuf0,
                   ebuf1, gbuf0, gbuf1, acc, gsem0, gsem1, ssem0, ssem1,
                   isem0, isem1):
    c = lax.axis_index("c")
    s = lax.axis_index("s")

    @pl.when(c == 0)
    def _():
        _hop32_half(ma_hbm, ta_hbm, s, e_hbm, z2_hbm, ebuf0, ebuf1, gbuf0,
                    gbuf1, acc, gsem0, gsem1, ssem0, ssem1, isem0, isem1)

    @pl.when(c == 1)
    def _():
        _hop32_half(mb_hbm, tb_hbm, s, e_hbm, z2_hbm, ebuf0, ebuf1, gbuf0,
                    gbuf1, acc, gsem0, gsem1, ssem0, ssem1, isem0, isem1)


_sc_hop32 = functools.partial(
    pl.kernel,
    out_type=(
        jax.ShapeDtypeStruct((NP, H), _f32),
        jax.ShapeDtypeStruct((NP, H), _f32),
    ),
    mesh=_MESH,
    compiler_params=pltpu.CompilerParams(needs_layout_passes=False, use_tc_tiling_on_sc=False),
    scratch_types=[
        pltpu.VMEM((GPC, 3, EPG), _i32),
        pltpu.VMEM((GPC, 3, EPG), _i32),
        pltpu.VMEM((CHUNK, H), _f32),
        pltpu.VMEM((CHUNK, H), _f32),
        pltpu.VMEM_SHARED((NP, H), _f32),
        pltpu.SemaphoreType.DMA,
        pltpu.SemaphoreType.DMA,
        pltpu.SemaphoreType.DMA,
        pltpu.SemaphoreType.DMA,
        pltpu.SemaphoreType.DMA,
        pltpu.SemaphoreType.DMA,
    ],
)(_sc_hop32_body)


def _sc_hop1_body(m_hbm, row_hbm, col_hbm, w_hbm, out_hbm, rbuf, cbufv,
                  wbuf, mloc, accl):
    c = lax.axis_index("c")
    s = lax.axis_index("s")
    wid = c * NSUB + s
    pltpu.sync_copy(m_hbm, mloc)
    _zero_local(accl)

    def chunk(ci, carry):
        base = wid * 25600 + ci * CH1
        pltpu.sync_copy(row_hbm.at[pl.ds(base, CH1)], rbuf)
        pltpu.sync_copy(col_hbm.at[pl.ds(base, CH1)], cbufv)
        pltpu.sync_copy(w_hbm.at[pl.ds(base, CH1)], wbuf)

        def blk(i, cc):
            rv = rbuf[pl.ds(i * 16, 16)]
            cv = cbufv[pl.ds(i * 16, 16)]
            wv = wbuf[pl.ds(i * 16, 16)]
            mv = plsc.load_gather(mloc, [rv])
            plsc.addupdate_scatter(accl, [cv], mv * wv)
            return cc

        lax.fori_loop(0, CH1 // 16, blk, jnp.int32(0))
        return carry

    lax.fori_loop(0, 25600 // CH1, chunk, jnp.int32(0))
    _w1_writeback(accl, out_hbm, wid)


_sc_hop1 = functools.partial(
    pl.kernel,
    out_type=jax.ShapeDtypeStruct((32, NP), _f32),
    mesh=_MESH,
    compiler_params=pltpu.CompilerParams(needs_layout_passes=False, use_tc_tiling_on_sc=False),
    scratch_types=[
        pltpu.VMEM((CH1,), _i32),
        pltpu.VMEM((CH1,), _i32),
        pltpu.VMEM((CH1,), _f32),
        pltpu.VMEM((NP,), _f32),
        pltpu.VMEM((NP,), _f32),
    ],
)(_sc_hop1_body)


def _t0_body(degp_ref, x_ref, w_ref, dis_ref, dis2_ref, ma_ref, mb_ref, oa_ref):
    deg = jnp.sum(degp_ref[...], axis=0)
    mask = deg > 0
    dis = jnp.where(mask, lax.rsqrt(deg), 0.0)
    dis2 = jnp.where(mask, 1.0 / deg, 0.0)
    dis_ref[...] = dis
    dis2_ref[...] = dis2
    x = x_ref[...]
    m0 = x * dis[:, None]
    ma_ref[...] = m0[:, :H]
    mb_ref[...] = m0[:, H:]
    oa_ref[...] = jnp.dot(x, w_ref[...], preferred_element_type=_f32)


def _t0(degp, x, w10):
    return pl.pallas_call(
        _t0_body,
        grid=(NB,),
        in_specs=[
            pl.BlockSpec((32, BN), lambda i: (0, i)),
            pl.BlockSpec((BN, F), lambda i: (i, 0)),
            pl.BlockSpec((F, F), lambda i: (0, 0)),
        ],
        out_specs=[
            pl.BlockSpec((BN,), lambda i: (i,)),
            pl.BlockSpec((BN,), lambda i: (i,)),
            pl.BlockSpec((BN, H), lambda i: (i, 0)),
            pl.BlockSpec((BN, H), lambda i: (i, 0)),
            pl.BlockSpec((BN, F), lambda i: (i, 0)),
        ],
        out_shape=[
            jax.ShapeDtypeStruct((NP,), _f32),
            jax.ShapeDtypeStruct((NP,), _f32),
            jax.ShapeDtypeStruct((NP, H), _f32),
            jax.ShapeDtypeStruct((NP, H), _f32),
            jax.ShapeDtypeStruct((NP, F), _f32),
        ],
    )(degp, x, w10)


def _thop_body(ta_ref, tb_ref, dis_ref, dis2_ref, oa_ref, w_ref, oao_ref,
               ma_ref, mb_ref):
    t = jnp.concatenate([ta_ref[...], tb_ref[...]], axis=1)
    dis = dis_ref[...]
    td = t * dis[:, None]
    oao_ref[...] = oa_ref[...] + jnp.dot(
        td, w_ref[...], preferred_element_type=_f32
    )
    m = t * dis2_ref[...][:, None]
    ma_ref[...] = m[:, :H]
    mb_ref[...] = m[:, H:]


def _thop(ta, tb, dis, dis2, oa, wk):
    return pl.pallas_call(
        _thop_body,
        grid=(NB,),
        in_specs=[
            pl.BlockSpec((BN, H), lambda i: (i, 0)),
            pl.BlockSpec((BN, H), lambda i: (i, 0)),
            pl.BlockSpec((BN,), lambda i: (i,)),
            pl.BlockSpec((BN,), lambda i: (i,)),
            pl.BlockSpec((BN, F), lambda i: (i, 0)),
            pl.BlockSpec((F, F), lambda i: (0, 0)),
        ],
        out_specs=[
            pl.BlockSpec((BN, F), lambda i: (i, 0)),
            pl.BlockSpec((BN, H), lambda i: (i, 0)),
            pl.BlockSpec((BN, H), lambda i: (i, 0)),
        ],
        out_shape=[
            jax.ShapeDtypeStruct((NP, F), _f32),
            jax.ShapeDtypeStruct((NP, H), _f32),
            jax.ShapeDtypeStruct((NP, H), _f32),
        ],
    )(ta, tb, dis, dis2, oa, wk)


def _tlend_body(ta_ref, tb_ref, dis_ref, oa_ref, w_ref, b_ref, wn_ref,
                oao_ref, ma_ref, mb_ref):
    t = jnp.concatenate([ta_ref[...], tb_ref[...]], axis=1)
    dis = dis_ref[...]
    td = t * dis[:, None]
    h = oa_ref[...] + jnp.dot(td, w_ref[...], preferred_element_type=_f32)
    h = jnp.maximum(h + b_ref[...][None, :], 0.0)
    oao_ref[...] = jnp.dot(h, wn_ref[...], preferred_element_type=_f32)
    m = h * dis[:, None]
    ma_ref[...] = m[:, :H]
    mb_ref[...] = m[:, H:]


def _tlend(ta, tb, dis, oa, wk, b, wn0):
    return pl.pallas_call(
        _tlend_body,
        grid=(NB,),
        in_specs=[
            pl.BlockSpec((BN, H), lambda i: (i, 0)),
            pl.BlockSpec((BN, H), lambda i: (i, 0)),
            pl.BlockSpec((BN,), lambda i: (i,)),
            pl.BlockSpec((BN, F), lambda i: (i, 0)),
            pl.BlockSpec((F, F), lambda i: (0, 0)),
            pl.BlockSpec((F,), lambda i: (0,)),
            pl.BlockSpec((F, F), lambda i: (0, 0)),
        ],
        out_specs=[
            pl.BlockSpec((BN, F), lambda i: (i, 0)),
            pl.BlockSpec((BN, H), lambda i: (i, 0)),
            pl.BlockSpec((BN, H), lambda i: (i, 0)),
        ],
        out_shape=[
            jax.ShapeDtypeStruct((NP, F), _f32),
            jax.ShapeDtypeStruct((NP, H), _f32),
            jax.ShapeDtypeStruct((NP, H), _f32),
        ],
    )(ta, tb, dis, oa, wk, b, wn0)


def _tl2end_body(ta_ref, tb_ref, dis_ref, oa_ref, w_ref, b_ref, w3_ref,
                 v_ref, m3_ref):
    t = jnp.concatenate([ta_ref[...], tb_ref[...]], axis=1)
    dis = dis_ref[...]
    td = t * dis[:, None]
    h = oa_ref[...] + jnp.dot(td, w_ref[...], preferred_element_type=_f32)
    h = jnp.maximum(h + b_ref[...][None, :], 0.0)
    v = jnp.dot(h, w3_ref[...], preferred_element_type=_f32)
    v_ref[...] = v
    m3_ref[...] = dis * v[:, 3]


def _tl2end(ta, tb, dis, oa, wk, b, w3c):
    return pl.pallas_call(
        _tl2end_body,
        grid=(NB,),
        in_specs=[
            pl.BlockSpec((BN, H), lambda i: (i, 0)),
            pl.BlockSpec((BN, H), lambda i: (i, 0)),
            pl.BlockSpec((BN,), lambda i: (i,)),
            pl.BlockSpec((BN, F), lambda i: (i, 0)),
            pl.BlockSpec((F, F), lambda i: (0, 0)),
            pl.BlockSpec((F,), lambda i: (0,)),
            pl.BlockSpec((F, 4), lambda i: (0, 0)),
        ],
        out_specs=[
            pl.BlockSpec((BN, 4), lambda i: (i, 0)),
            pl.BlockSpec((BN,), lambda i: (i,)),
        ],
        out_shape=[
            jax.ShapeDtypeStruct((NP, 4), _f32),
            jax.ShapeDtypeStruct((NP,), _f32),
        ],
    )(ta, tb, dis, oa, wk, b, w3c)


def _tw1_body(p_ref, v_ref, dis_ref, dis2_ref, m_ref, *, k):
    t = jnp.sum(p_ref[...], axis=0)
    m_ref[...] = dis_ref[...] * v_ref[:, k] + dis2_ref[...] * t


def _tw1(p, v, dis, dis2, k):
    return pl.pallas_call(
        functools.partial(_tw1_body, k=k),
        grid=(NB,),
        in_specs=[
            pl.BlockSpec((32, BN), lambda i: (0, i)),
            pl.BlockSpec((BN, 4), lambda i: (i, 0)),
            pl.BlockSpec((BN,), lambda i: (i,)),
            pl.BlockSpec((BN,), lambda i: (i,)),
        ],
        out_specs=pl.BlockSpec((BN,), lambda i: (i,)),
        out_shape=jax.ShapeDtypeStruct((NP,), _f32),
    )(p, v, dis, dis2)


def _tfinal_body(p_ref, v_ref, dis_ref, b3_ref, batch_ref, y_ref):
    i = pl.program_id(0)

    @pl.when(i == 0)
    def _():
        y_ref[...] = jnp.zeros_like(y_ref)

    out3 = v_ref[:, 0] + dis_ref[...] * jnp.sum(p_ref[...], axis=0) + b3_ref[0]
    b = batch_ref[0]
    onehot = (
        b[None, :] == lax.broadcasted_iota(jnp.int32, (G, 1), 0)
    ).astype(_f32)
    y_ref[...] += onehot @ out3[:, None]

    @pl.when(i == NB - 1)
    def _():
        y_ref[...] = jax.nn.sigmoid(y_ref[...])


def _tfinal(p, v, dis, b3, batch2d):
    return pl.pallas_call(
        _tfinal_body,
        grid=(NB,),
        in_specs=[
            pl.BlockSpec((32, BN), lambda i: (0, i)),
            pl.BlockSpec((BN, 4), lambda i: (i, 0)),
            pl.BlockSpec((BN,), lambda i: (i,)),
            pl.BlockSpec((1,), lambda i: (0,)),
            pl.BlockSpec((1, BN), lambda i: (0, i)),
        ],
        out_specs=pl.BlockSpec((G, 1), lambda i: (0, 0)),
        out_shape=jax.ShapeDtypeStruct((G, 1), _f32),
    )(p, v, dis, b3, batch2d)


def kernel(x, batch, edge_index, edge_weight, W1, b1, W2, b2, W3, b3):
    row = edge_index[0]
    col = edge_index[1]
    rowp = jnp.concatenate([row, jnp.zeros((EP - E,), _i32)])
    colp = jnp.concatenate([col, jnp.zeros((EP - E,), _i32)])
    wp = jnp.concatenate([edge_weight, jnp.zeros((EP - E,), _f32)])
    col2d = colp.reshape(EP // EPG, EPG)
    epack = jnp.stack(
        [rowp.reshape(EP // EPG, EPG), col2d,
         wp.view(_i32).reshape(EP // EPG, EPG)], axis=1,
    )
    x_pad = jnp.zeros((NP, F), _f32).at[:N].set(x)
    batch2d = jnp.full((NP,), -1, _i32).at[:N].set(batch).reshape(1, NP)
    z2 = jnp.zeros((RPS, H), _f32)
    w3c = jnp.transpose(W3[:, :, 0])

    degp = _sc_deg(colp, wp)
    dis, dis2, ma, mb, oa = _t0(degp, x_pad, W1[0])

    for layer in range(2):
        Wl = W1 if layer == 0 else W2
        for k in (1, 2):
            ta, tb = _sc_hop32(ma, mb, epack, z2)
            oa, ma, mb = _thop(ta, tb, dis, dis2, oa, Wl[k])
        ta, tb = _sc_hop32(ma, mb, epack, z2)
        if layer == 0:
            oa, ma, mb = _tlend(ta, tb, dis, oa, W1[3], b1, W2[0])
        else:
            v, m = _tl2end(ta, tb, dis, oa, W2[3], b2, w3c)

    for k in (2, 1):
        p = _sc_hop1(m, rowp, colp, wp)
        m = _tw1(p, v, dis, dis2, k)
    p = _sc_hop1(m, rowp, colp, wp)
    return _tfinal(p, v, dis, b3, batch2d)


# speedup vs baseline: 1.1320x; 1.1320x over previous
import functools

import jax
import jax.numpy as jnp
from jax import lax
from jax.experimental import pallas as pl
from jax.experimental.pallas import tpu as pltpu
from jax.experimental.pallas import tpu_sc as plsc

N = 50000
E = 800000
G = 32
F = 64
H = 32

NP = 51200
EP = 819200
EPG = 128
GPC = 2
CHUNK = EPG * GPC
NSUB = 16
RPS = NP // NSUB
BN = 2048
NB = NP // BN

_MESH = plsc.VectorSubcoreMesh(core_axis_name="c", subcore_axis_name="s")
_f32 = jnp.float32
_i32 = jnp.int32


def _zero_slice(zsrc, acc, s):
    pltpu.sync_copy(zsrc, acc.at[pl.ds(s * RPS, RPS)])


CH1 = 3200


def _zero_local(accl):
    def zb(i, cc):
        accl[pl.ds(i * 16, 16)] = jnp.zeros((16,), _f32)
        return cc
    lax.fori_loop(0, NP // 16, zb, jnp.int32(0))


def _w1_writeback(accl, out_hbm, wid):
    pltpu.sync_copy(accl, out_hbm.at[wid])


def _sc_deg_body(col_hbm, w_hbm, out_hbm, cbufv, wbuf, accl):
    c = lax.axis_index("c")
    s = lax.axis_index("s")
    wid = c * NSUB + s
    _zero_local(accl)

    def chunk(ci, carry):
        base = wid * 25600 + ci * CH1
        pltpu.sync_copy(col_hbm.at[pl.ds(base, CH1)], cbufv)
        pltpu.sync_copy(w_hbm.at[pl.ds(base, CH1)], wbuf)

        def blk(i, cc):
            cv = cbufv[pl.ds(i * 16, 16)]
            wv = wbuf[pl.ds(i * 16, 16)]
            plsc.addupdate_scatter(accl, [cv], wv)
            return cc

        lax.fori_loop(0, CH1 // 16, blk, jnp.int32(0))
        return carry

    lax.fori_loop(0, 25600 // CH1, chunk, jnp.int32(0))
    _w1_writeback(accl, out_hbm, wid)


_sc_deg = functools.partial(
    pl.kernel,
    out_type=jax.ShapeDtypeStruct((32, NP), _f32),
    mesh=_MESH,
    compiler_params=pltpu.CompilerParams(needs_layout_passes=False, use_tc_tiling_on_sc=False),
    scratch_types=[
        pltpu.VMEM((CH1,), _i32),
        pltpu.VMEM((CH1,), _f32),
        pltpu.VMEM((NP,), _f32),
    ],
)(_sc_deg_body)


def _hop32_gather(m_hbm, ebuf, gbuf, sem, gbase):
    return [
        pltpu.async_copy(
            m_hbm.at[ebuf.at[g, 0]], gbuf.at[pl.ds(g * EPG, EPG)], sem
        )
        for g in range(GPC)
    ]


def _hop32_drain_gather(m_hbm, ebuf, gbuf, sem):
    for g in range(GPC):
        pltpu.make_async_copy(
            m_hbm.at[ebuf.at[g, 0]], gbuf.at[pl.ds(g * EPG, EPG)], sem
        ).wait()


def _hop32_scale(ebuf, gbuf):
    for g in range(GPC):
        def blk(i, cc, g=g):
            wv = plsc.bitcast(ebuf[g, 2, pl.ds(i * 16, 16)], _f32)
            for jj in range(16):
                e = g * EPG + i * 16 + jj
                sp = lax.gather(
                    wv, jnp.full((16, 1), jj, _i32),
                    lax.GatherDimensionNumbers(
                        offset_dims=(), collapsed_slice_dims=(0,),
                        start_index_map=(0,)),
                    (1,), mode=lax.GatherScatterMode.PROMISE_IN_BOUNDS)
                gbuf[e, pl.ds(0, 16)] = gbuf[e, pl.ds(0, 16)] * sp
                gbuf[e, pl.ds(16, 16)] = gbuf[e, pl.ds(16, 16)] * sp
            return cc
        lax.fori_loop(0, EPG // 16, blk, jnp.int32(0))


def _hop32_scatter(acc, ebuf, gbuf, sem):
    return [
        pltpu.async_copy(
            gbuf.at[pl.ds(g * EPG, EPG)], acc.at[ebuf.at[g, 1]], sem, add=True
        )
        for g in range(GPC)
    ]


def _hop32_half(m_hbm, t_hbm, mo_hbm, s, e_hbm, z2_hbm, dbuf, ebuf0, ebuf1,
                gbuf0, gbuf1, acc, gsem0, gsem1, ssem0, ssem1, isem0, isem1):
    nch = 400 // GPC
    gps = s * 400

    pltpu.sync_copy(e_hbm.at[pl.ds(gps, GPC)], ebuf0)
    pltpu.sync_copy(e_hbm.at[pl.ds(gps + GPC, GPC)], ebuf1)
    _hop32_gather(m_hbm, ebuf0, gbuf0, gsem0, 0)
    _hop32_gather(m_hbm, ebuf1, gbuf1, gsem1, 0)

    def body(i, carry):
        p0 = jnp.minimum(2 * i + 2, nch - 1)
        p1 = jnp.minimum(2 * i + 3, nch - 1)
        _hop32_drain_gather(m_hbm, ebuf0, gbuf0, gsem0)
        _hop32_scale(ebuf0, gbuf0)
        s0 = _hop32_scatter(acc, ebuf0, gbuf0, ssem0)
        _hop32_drain_gather(m_hbm, ebuf1, gbuf1, gsem1)
        _hop32_scale(ebuf1, gbuf1)
        s1 = _hop32_scatter(acc, ebuf1, gbuf1, ssem1)
        for d in s0:
            d.wait()
        i0 = pltpu.async_copy(e_hbm.at[pl.ds(gps + p0 * GPC, GPC)], ebuf0,
                              isem0)
        for d in s1:
            d.wait()
        i1 = pltpu.async_copy(e_hbm.at[pl.ds(gps + p1 * GPC, GPC)], ebuf1,
                              isem1)
        i0.wait()
        _hop32_gather(m_hbm, ebuf0, gbuf0, gsem0, 0)
        i1.wait()
        _hop32_gather(m_hbm, ebuf1, gbuf1, gsem1, 0)
        return carry

    lax.fori_loop(0, nch // 2, body, jnp.int32(0))
    _hop32_drain_gather(m_hbm, ebuf0, gbuf0, gsem0)
    _hop32_drain_gather(m_hbm, ebuf1, gbuf1, gsem1)
    plsc.subcore_barrier()
    pltpu.sync_copy(acc.at[pl.ds(s * RPS, RPS)], t_hbm.at[pl.ds(s * RPS, RPS)])
    if mo_hbm is not None:
        base = s * RPS
        off = 0
        for rows in [256] * 12 + [128]:
            pltpu.sync_copy(acc.at[pl.ds(base + off, rows)],
                            gbuf0.at[pl.ds(0, rows)])

            def rblk(i, cc, off=off):
                dv = dbuf[pl.ds(off + i * 16, 16)]
                for jj in range(16):
                    r = i * 16 + jj
                    sp = lax.gather(
                        dv, jnp.full((16, 1), jj, _i32),
                        lax.GatherDimensionNumbers(
                            offset_dims=(), collapsed_slice_dims=(0,),
                            start_index_map=(0,)),
                        (1,), mode=lax.GatherScatterMode.PROMISE_IN_BOUNDS)
                    gbuf0[r, pl.ds(0, 16)] = gbuf0[r, pl.ds(0, 16)] * sp
                    gbuf0[r, pl.ds(16, 16)] = gbuf0[r, pl.ds(16, 16)] * sp
                return cc

            lax.fori_loop(0, rows // 16, rblk, jnp.int32(0))
            pltpu.sync_copy(gbuf0.at[pl.ds(0, rows)],
                            mo_hbm.at[pl.ds(base + off, rows)])
            off += rows
    pltpu.sync_copy(z2_hbm, acc.at[pl.ds(s * RPS, RPS)])
    plsc.subcore_barrier()


def _sc_layer_body(ma_hbm, mb_hbm, e_hbm, z2_hbm, dis2_hbm, t1a, t1b, t2a,
                   t2b, t3a, t3b, m1a, m1b, m2a, m2b, dbuf, ebuf0, ebuf1,
                   gbuf0, gbuf1, acc, gsem0, gsem1, ssem0, ssem1, isem0,
                   isem1):
    c = lax.axis_index("c")
    s = lax.axis_index("s")
    pltpu.sync_copy(dis2_hbm.at[pl.ds(s * RPS, RPS)], dbuf)
    pltpu.sync_copy(z2_hbm, acc.at[pl.ds(s * RPS, RPS)])
    plsc.subcore_barrier()

    @pl.when(c == 0)
    def _():
        for m_in, t_out, m_out in ((ma_hbm, t1a, m1a), (m1a, t2a, m2a),
                                   (m2a, t3a, None)):
            _hop32_half(m_in, t_out, m_out, s, e_hbm, z2_hbm, dbuf, ebuf0,
                        ebuf1, gbuf0, gbuf1, acc, gsem0, gsem1, ssem0, ssem1,
                        isem0, isem1)

    @pl.when(c == 1)
    def _():
        for m_in, t_out, m_out in ((mb_hbm, t1b, m1b), (m1b, t2b, m2b),
                                   (m2b, t3b, None)):
            _hop32_half(m_in, t_out, m_out, s, e_hbm, z2_hbm, dbuf, ebuf0,
                        ebuf1, gbuf0, gbuf1, acc, gsem0, gsem1, ssem0, ssem1,
                        isem0, isem1)


_sc_layer = functools.partial(
    pl.kernel,
    out_type=tuple(
        jax.ShapeDtypeStruct((NP, H), _f32) for _ in range(10)
    ),
    mesh=_MESH,
    compiler_params=pltpu.CompilerParams(needs_layout_passes=False, use_tc_tiling_on_sc=False),
    scratch_types=[
        pltpu.VMEM((RPS,), _f32),
        pltpu.VMEM((GPC, 3, EPG), _i32),
        pltpu.VMEM((GPC, 3, EPG), _i32),
        pltpu.VMEM((CHUNK, H), _f32),
        pltpu.VMEM((CHUNK, H), _f32),
        pltpu.VMEM_SHARED((NP, H), _f32),
        pltpu.SemaphoreType.DMA,
        pltpu.SemaphoreType.DMA,
        pltpu.SemaphoreType.DMA,
        pltpu.SemaphoreType.DMA,
        pltpu.SemaphoreType.DMA,
        pltpu.SemaphoreType.DMA,
    ],
)(_sc_layer_body)


def _sc_hop1_body(m_hbm, row_hbm, col_hbm, w_hbm, out_hbm, rbuf, cbufv,
                  wbuf, mloc, accl):
    c = lax.axis_index("c")
    s = lax.axis_index("s")
    wid = c * NSUB + s
    pltpu.sync_copy(m_hbm, mloc)
    _zero_local(accl)

    def chunk(ci, carry):
        base = wid * 25600 + ci * CH1
        pltpu.sync_copy(row_hbm.at[pl.ds(base, CH1)], rbuf)
        pltpu.sync_copy(col_hbm.at[pl.ds(base, CH1)], cbufv)
        pltpu.sync_copy(w_hbm.at[pl.ds(base, CH1)], wbuf)

        def blk(i, cc):
            rv = rbuf[pl.ds(i * 16, 16)]
            cv = cbufv[pl.ds(i * 16, 16)]
            wv = wbuf[pl.ds(i * 16, 16)]
            mv = plsc.load_gather(mloc, [rv])
            plsc.addupdate_scatter(accl, [cv], mv * wv)
            return cc

        lax.fori_loop(0, CH1 // 16, blk, jnp.int32(0))
        return carry

    lax.fori_loop(0, 25600 // CH1, chunk, jnp.int32(0))
    _w1_writeback(accl, out_hbm, wid)


_sc_hop1 = functools.partial(
    pl.kernel,
    out_type=jax.ShapeDtypeStruct((32, NP), _f32),
    mesh=_MESH,
    compiler_params=pltpu.CompilerParams(needs_layout_passes=False, use_tc_tiling_on_sc=False),
    scratch_types=[
        pltpu.VMEM((CH1,), _i32),
        pltpu.VMEM((CH1,), _i32),
        pltpu.VMEM((CH1,), _f32),
        pltpu.VMEM((NP,), _f32),
        pltpu.VMEM((NP,), _f32),
    ],
)(_sc_hop1_body)


def _t0_body(degp_ref, x_ref, w_ref, dis_ref, dis2_ref, ma_ref, mb_ref, oa_ref):
    deg = jnp.sum(degp_ref[...], axis=0)
    mask = deg > 0
    dis = jnp.where(mask, lax.rsqrt(deg), 0.0)
    dis2 = jnp.where(mask, 1.0 / deg, 0.0)
    dis_ref[...] = dis
    dis2_ref[...] = dis2
    x = x_ref[...]
    m0 = x * dis[:, None]
    ma_ref[...] = m0[:, :H]
    mb_ref[...] = m0[:, H:]
    oa_ref[...] = jnp.dot(x, w_ref[...], preferred_element_type=_f32)


def _t0(degp, x, w10):
    return pl.pallas_call(
        _t0_body,
        grid=(NB,),
        in_specs=[
            pl.BlockSpec((32, BN), lambda i: (0, i)),
            pl.BlockSpec((BN, F), lambda i: (i, 0)),
            pl.BlockSpec((F, F), lambda i: (0, 0)),
        ],
        out_specs=[
            pl.BlockSpec((BN,), lambda i: (i,)),
            pl.BlockSpec((BN,), lambda i: (i,)),
            pl.BlockSpec((BN, H), lambda i: (i, 0)),
            pl.BlockSpec((BN, H), lambda i: (i, 0)),
            pl.BlockSpec((BN, F), lambda i: (i, 0)),
        ],
        out_shape=[
            jax.ShapeDtypeStruct((NP,), _f32),
            jax.ShapeDtypeStruct((NP,), _f32),
            jax.ShapeDtypeStruct((NP, H), _f32),
            jax.ShapeDtypeStruct((NP, H), _f32),
            jax.ShapeDtypeStruct((NP, F), _f32),
        ],
    )(degp, x, w10)


def _thop_body(ta_ref, tb_ref, dis_ref, oa_ref, w_ref, oao_ref):
    t = jnp.concatenate([ta_ref[...], tb_ref[...]], axis=1)
    td = t * dis_ref[...][:, None]
    oao_ref[...] = oa_ref[...] + jnp.dot(
        td, w_ref[...], preferred_element_type=_f32
    )


def _thop(ta, tb, dis, oa, wk):
    return pl.pallas_call(
        _thop_body,
        grid=(NB,),
        in_specs=[
            pl.BlockSpec((BN, H), lambda i: (i, 0)),
            pl.BlockSpec((BN, H), lambda i: (i, 0)),
            pl.BlockSpec((BN,), lambda i: (i,)),
            pl.BlockSpec((BN, F), lambda i: (i, 0)),
            pl.BlockSpec((F, F), lambda i: (0, 0)),
        ],
        out_specs=pl.BlockSpec((BN, F), lambda i: (i, 0)),
        out_shape=jax.ShapeDtypeStruct((NP, F), _f32),
    )(ta, tb, dis, oa, wk)


def _tlend_body(ta_ref, tb_ref, dis_ref, oa_ref, w_ref, b_ref, wn_ref,
                oao_ref, ma_ref, mb_ref):
    t = jnp.concatenate([ta_ref[...], tb_ref[...]], axis=1)
    dis = dis_ref[...]
    td = t * dis[:, None]
    h = oa_ref[...] + jnp.dot(td, w_ref[...], preferred_element_type=_f32)
    h = jnp.maximum(h + b_ref[...][None, :], 0.0)
    oao_ref[...] = jnp.dot(h, wn_ref[...], preferred_element_type=_f32)
    m = h * dis[:, None]
    ma_ref[...] = m[:, :H]
    mb_ref[...] = m[:, H:]


def _tlend(ta, tb, dis, oa, wk, b, wn0):
    return pl.pallas_call(
        _tlend_body,
        grid=(NB,),
        in_specs=[
            pl.BlockSpec((BN, H), lambda i: (i, 0)),
            pl.BlockSpec((BN, H), lambda i: (i, 0)),
            pl.BlockSpec((BN,), lambda i: (i,)),
            pl.BlockSpec((BN, F), lambda i: (i, 0)),
            pl.BlockSpec((F, F), lambda i: (0, 0)),
            pl.BlockSpec((F,), lambda i: (0,)),
            pl.BlockSpec((F, F), lambda i: (0, 0)),
        ],
        out_specs=[
            pl.BlockSpec((BN, F), lambda i: (i, 0)),
            pl.BlockSpec((BN, H), lambda i: (i, 0)),
            pl.BlockSpec((BN, H), lambda i: (i, 0)),
        ],
        out_shape=[
            jax.ShapeDtypeStruct((NP, F), _f32),
            jax.ShapeDtypeStruct((NP, H), _f32),
            jax.ShapeDtypeStruct((NP, H), _f32),
        ],
    )(ta, tb, dis, oa, wk, b, wn0)


def _tl2end_body(ta_ref, tb_ref, dis_ref, oa_ref, w_ref, b_ref, w3_ref,
                 v_ref, m3_ref):
    t = jnp.concatenate([ta_ref[...], tb_ref[...]], axis=1)
    dis = dis_ref[...]
    td = t * dis[:, None]
    h = oa_ref[...] + jnp.dot(td, w_ref[...], preferred_element_type=_f32)
    h = jnp.maximum(h + b_ref[...][None, :], 0.0)
    v = jnp.dot(h, w3_ref[...], preferred_element_type=_f32)
    v_ref[...] = v
    m3_ref[...] = dis * v[:, 3]


def _tl2end(ta, tb, dis, oa, wk, b, w3c):
    return pl.pallas_call(
        _tl2end_body,
        grid=(NB,),
        in_specs=[
            pl.BlockSpec((BN, H), lambda i: (i, 0)),
            pl.BlockSpec((BN, H), lambda i: (i, 0)),
            pl.BlockSpec((BN,), lambda i: (i,)),
            pl.BlockSpec((BN, F), lambda i: (i, 0)),
            pl.BlockSpec((F, F), lambda i: (0, 0)),
            pl.BlockSpec((F,), lambda i: (0,)),
            pl.BlockSpec((F, 4), lambda i: (0, 0)),
        ],
        out_specs=[
            pl.BlockSpec((BN, 4), lambda i: (i, 0)),
            pl.BlockSpec((BN,), lambda i: (i,)),
        ],
        out_shape=[
            jax.ShapeDtypeStruct((NP, 4), _f32),
            jax.ShapeDtypeStruct((NP,), _f32),
        ],
    )(ta, tb, dis, oa, wk, b, w3c)


def _tw1_body(p_ref, v_ref, dis_ref, dis2_ref, m_ref, *, k):
    t = jnp.sum(p_ref[...], axis=0)
    m_ref[...] = dis_ref[...] * v_ref[:, k] + dis2_ref[...] * t


def _tw1(p, v, dis, dis2, k):
    return pl.pallas_call(
        functools.partial(_tw1_body, k=k),
        grid=(NB,),
        in_specs=[
            pl.BlockSpec((32, BN), lambda i: (0, i)),
            pl.BlockSpec((BN, 4), lambda i: (i, 0)),
            pl.BlockSpec((BN,), lambda i: (i,)),
            pl.BlockSpec((BN,), lambda i: (i,)),
        ],
        out_specs=pl.BlockSpec((BN,), lambda i: (i,)),
        out_shape=jax.ShapeDtypeStruct((NP,), _f32),
    )(p, v, dis, dis2)


def _tfinal_body(p_ref, v_ref, dis_ref, b3_ref, batch_ref, y_ref):
    i = pl.program_id(0)

    @pl.when(i == 0)
    def _():
        y_ref[...] = jnp.zeros_like(y_ref)

    out3 = v_ref[:, 0] + dis_ref[...] * jnp.sum(p_ref[...], axis=0) + b3_ref[0]
    b = batch_ref[0]
    onehot = (
        b[None, :] == lax.broadcasted_iota(jnp.int32, (G, 1), 0)
    ).astype(_f32)
    y_ref[...] += onehot @ out3[:, None]

    @pl.when(i == NB - 1)
    def _():
        y_ref[...] = jax.nn.sigmoid(y_ref[...])


def _tfinal(p, v, dis, b3, batch2d):
    return pl.pallas_call(
        _tfinal_body,
        grid=(NB,),
        in_specs=[
            pl.BlockSpec((32, BN), lambda i: (0, i)),
            pl.BlockSpec((BN, 4), lambda i: (i, 0)),
            pl.BlockSpec((BN,), lambda i: (i,)),
            pl.BlockSpec((1,), lambda i: (0,)),
            pl.BlockSpec((1, BN), lambda i: (0, i)),
        ],
        out_specs=pl.BlockSpec((G, 1), lambda i: (0, 0)),
        out_shape=jax.ShapeDtypeStruct((G, 1), _f32),
    )(p, v, dis, b3, batch2d)


def kernel(x, batch, edge_index, edge_weight, W1, b1, W2, b2, W3, b3):
    row = edge_index[0]
    col = edge_index[1]
    rowp = jnp.concatenate([row, jnp.zeros((EP - E,), _i32)])
    colp = jnp.concatenate([col, jnp.zeros((EP - E,), _i32)])
    wp = jnp.concatenate([edge_weight, jnp.zeros((EP - E,), _f32)])
    col2d = colp.reshape(EP // EPG, EPG)
    epack = jnp.stack(
        [rowp.reshape(EP // EPG, EPG), col2d,
         wp.view(_i32).reshape(EP // EPG, EPG)], axis=1,
    )
    x_pad = jnp.zeros((NP, F), _f32).at[:N].set(x)
    batch2d = jnp.full((NP,), -1, _i32).at[:N].set(batch).reshape(1, NP)
    z2 = jnp.zeros((RPS, H), _f32)
    w3c = jnp.transpose(W3[:, :, 0])

    degp = _sc_deg(colp, wp)
    dis, dis2, ma, mb, oa = _t0(degp, x_pad, W1[0])

    for layer in range(2):
        Wl = W1 if layer == 0 else W2
        t1a, t1b, t2a, t2b, t3a, t3b = _sc_layer(ma, mb, epack, z2, dis2)[:6]
        oa = _thop(t1a, t1b, dis, oa, Wl[1])
        oa = _thop(t2a, t2b, dis, oa, Wl[2])
        if layer == 0:
            oa, ma, mb = _tlend(t3a, t3b, dis, oa, W1[3], b1, W2[0])
        else:
            v, m = _tl2end(t3a, t3b, dis, oa, W2[3], b2, w3c)

    for k in (2, 1):
        p = _sc_hop1(m, rowp, colp, wp)
        m = _tw1(p, v, dis, dis2, k)
    p = _sc_hop1(m, rowp, colp, wp)
    return _tfinal(p, v, dis, b3, batch2d)


# speedup vs baseline: 1.1356x; 1.0032x over previous
import functools

import jax
import jax.numpy as jnp
from jax import lax
from jax.experimental import pallas as pl
from jax.experimental.pallas import tpu as pltpu
from jax.experimental.pallas import tpu_sc as plsc

N = 50000
E = 800000
G = 32
F = 64
H = 32

NP = 51200
EP = 819200
EPG = 128
GPC = 2
CHUNK = EPG * GPC
NSUB = 16
RPS = NP // NSUB
BN = 2048
NB = NP // BN

_MESH = plsc.VectorSubcoreMesh(core_axis_name="c", subcore_axis_name="s")
_f32 = jnp.float32
_i32 = jnp.int32


def _zero_slice(zsrc, acc, s):
    pltpu.sync_copy(zsrc, acc.at[pl.ds(s * RPS, RPS)])


CH1 = 3200


def _zero_local(accl):
    def zb(i, cc):
        accl[pl.ds(i * 16, 16)] = jnp.zeros((16,), _f32)
        return cc
    lax.fori_loop(0, NP // 16, zb, jnp.int32(0))


def _w1_writeback(accl, out_hbm, wid):
    pltpu.sync_copy(accl, out_hbm.at[wid])


def _sc_deg_body(col_hbm, w_hbm, out_hbm, cbufv, wbuf, accl):
    c = lax.axis_index("c")
    s = lax.axis_index("s")
    wid = c * NSUB + s
    _zero_local(accl)

    def chunk(ci, carry):
        base = wid * 25600 + ci * CH1
        pltpu.sync_copy(col_hbm.at[pl.ds(base, CH1)], cbufv)
        pltpu.sync_copy(w_hbm.at[pl.ds(base, CH1)], wbuf)

        def blk(i, cc):
            cv = cbufv[pl.ds(i * 16, 16)]
            wv = wbuf[pl.ds(i * 16, 16)]
            plsc.addupdate_scatter(accl, [cv], wv)
            return cc

        lax.fori_loop(0, CH1 // 16, blk, jnp.int32(0))
        return carry

    lax.fori_loop(0, 25600 // CH1, chunk, jnp.int32(0))
    _w1_writeback(accl, out_hbm, wid)


_sc_deg = functools.partial(
    pl.kernel,
    out_type=jax.ShapeDtypeStruct((32, NP), _f32),
    mesh=_MESH,
    compiler_params=pltpu.CompilerParams(needs_layout_passes=False, use_tc_tiling_on_sc=False),
    scratch_types=[
        pltpu.VMEM((CH1,), _i32),
        pltpu.VMEM((CH1,), _f32),
        pltpu.VMEM((NP,), _f32),
    ],
)(_sc_deg_body)


def _hop32_gather(m_hbm, ebuf, gbuf, sem, gbase):
    return [
        pltpu.async_copy(
            m_hbm.at[ebuf.at[g, 0]], gbuf.at[pl.ds(g * EPG, EPG)], sem
        )
        for g in range(GPC)
    ]


def _hop32_drain_gather(m_hbm, ebuf, gbuf, sem):
    for g in range(GPC):
        pltpu.make_async_copy(
            m_hbm.at[ebuf.at[g, 0]], gbuf.at[pl.ds(g * EPG, EPG)], sem
        ).wait()


def _hop32_scale(ebuf, gbuf):
    for g in range(GPC):
        def blk(i, cc, g=g):
            wv = plsc.bitcast(ebuf[g, 2, pl.ds(i * 16, 16)], _f32)
            for jj in range(16):
                e = g * EPG + i * 16 + jj
                sp = lax.gather(
                    wv, jnp.full((16, 1), jj, _i32),
                    lax.GatherDimensionNumbers(
                        offset_dims=(), collapsed_slice_dims=(0,),
                        start_index_map=(0,)),
                    (1,), mode=lax.GatherScatterMode.PROMISE_IN_BOUNDS)
                gbuf[e, pl.ds(0, 16)] = gbuf[e, pl.ds(0, 16)] * sp
                gbuf[e, pl.ds(16, 16)] = gbuf[e, pl.ds(16, 16)] * sp
            return cc
        lax.fori_loop(0, EPG // 16, blk, jnp.int32(0))


def _hop32_scatter(acc, ebuf, gbuf, sem):
    return [
        pltpu.async_copy(
            gbuf.at[pl.ds(g * EPG, EPG)], acc.at[ebuf.at[g, 1]], sem, add=True
        )
        for g in range(GPC)
    ]


def _hop32_half(m_hbm, t_hbm, mo_hbm, s, e_hbm, z2_hbm, dbuf, ebuf0, ebuf1,
                gbuf0, gbuf1, acc, gsem0, gsem1, ssem0, ssem1, isem0, isem1):
    nch = 400 // GPC
    gps = s * 400

    pltpu.sync_copy(e_hbm.at[pl.ds(gps, GPC)], ebuf0)
    pltpu.sync_copy(e_hbm.at[pl.ds(gps + GPC, GPC)], ebuf1)
    _hop32_gather(m_hbm, ebuf0, gbuf0, gsem0, 0)
    _hop32_gather(m_hbm, ebuf1, gbuf1, gsem1, 0)

    def body(i, carry):
        p0 = jnp.minimum(2 * i + 2, nch - 1)
        p1 = jnp.minimum(2 * i + 3, nch - 1)
        _hop32_drain_gather(m_hbm, ebuf0, gbuf0, gsem0)
        _hop32_scale(ebuf0, gbuf0)
        s0 = _hop32_scatter(acc, ebuf0, gbuf0, ssem0)
        _hop32_drain_gather(m_hbm, ebuf1, gbuf1, gsem1)
        _hop32_scale(ebuf1, gbuf1)
        s1 = _hop32_scatter(acc, ebuf1, gbuf1, ssem1)
        for d in s0:
            d.wait()
        i0 = pltpu.async_copy(e_hbm.at[pl.ds(gps + p0 * GPC, GPC)], ebuf0,
                              isem0)
        for d in s1:
            d.wait()
        i1 = pltpu.async_copy(e_hbm.at[pl.ds(gps + p1 * GPC, GPC)], ebuf1,
                              isem1)
        i0.wait()
        _hop32_gather(m_hbm, ebuf0, gbuf0, gsem0, 0)
        i1.wait()
        _hop32_gather(m_hbm, ebuf1, gbuf1, gsem1, 0)
        return carry

    lax.fori_loop(0, nch // 2, body, jnp.int32(0))
    _hop32_drain_gather(m_hbm, ebuf0, gbuf0, gsem0)
    _hop32_drain_gather(m_hbm, ebuf1, gbuf1, gsem1)
    plsc.subcore_barrier()
    pltpu.sync_copy(acc.at[pl.ds(s * RPS, RPS)], t_hbm.at[pl.ds(s * RPS, RPS)])
    if mo_hbm is not None:
        base = s * RPS
        off = 0
        for rows in [256] * 12 + [128]:
            pltpu.sync_copy(acc.at[pl.ds(base + off, rows)],
                            gbuf0.at[pl.ds(0, rows)])

            def rblk(i, cc, off=off):
                dv = dbuf[pl.ds(off + i * 16, 16)]
                for jj in range(16):
                    r = i * 16 + jj
                    sp = lax.gather(
                        dv, jnp.full((16, 1), jj, _i32),
                        lax.GatherDimensionNumbers(
                            offset_dims=(), collapsed_slice_dims=(0,),
                            start_index_map=(0,)),
                        (1,), mode=lax.GatherScatterMode.PROMISE_IN_BOUNDS)
                    gbuf0[r, pl.ds(0, 16)] = gbuf0[r, pl.ds(0, 16)] * sp
                    gbuf0[r, pl.ds(16, 16)] = gbuf0[r, pl.ds(16, 16)] * sp
                return cc

            lax.fori_loop(0, rows // 16, rblk, jnp.int32(0))
            pltpu.sync_copy(gbuf0.at[pl.ds(0, rows)],
                            mo_hbm.at[pl.ds(base + off, rows)])
            off += rows
    pltpu.sync_copy(z2_hbm, acc.at[pl.ds(s * RPS, RPS)])
    plsc.subcore_barrier()


def _sc_layer_body(ma_hbm, mb_hbm, e_hbm, z2_hbm, dis2_hbm, t1a, t1b, t2a,
                   t2b, t3a, t3b, m1a, m1b, m2a, m2b, dbuf, ebuf0, ebuf1,
                   gbuf0, gbuf1, acc, gsem0, gsem1, ssem0, ssem1, isem0,
                   isem1):
    c = lax.axis_index("c")
    s = lax.axis_index("s")
    pltpu.sync_copy(dis2_hbm.at[pl.ds(s * RPS, RPS)], dbuf)
    pltpu.sync_copy(z2_hbm, acc.at[pl.ds(s * RPS, RPS)])
    plsc.subcore_barrier()

    @pl.when(c == 0)
    def _():
        for m_in, t_out, m_out in ((ma_hbm, t1a, m1a), (m1a, t2a, m2a),
                                   (m2a, t3a, None)):
            _hop32_half(m_in, t_out, m_out, s, e_hbm, z2_hbm, dbuf, ebuf0,
                        ebuf1, gbuf0, gbuf1, acc, gsem0, gsem1, ssem0, ssem1,
                        isem0, isem1)

    @pl.when(c == 1)
    def _():
        for m_in, t_out, m_out in ((mb_hbm, t1b, m1b), (m1b, t2b, m2b),
                                   (m2b, t3b, None)):
            _hop32_half(m_in, t_out, m_out, s, e_hbm, z2_hbm, dbuf, ebuf0,
                        ebuf1, gbuf0, gbuf1, acc, gsem0, gsem1, ssem0, ssem1,
                        isem0, isem1)


_sc_layer = functools.partial(
    pl.kernel,
    out_type=tuple(
        jax.ShapeDtypeStruct((NP, H), _f32) for _ in range(10)
    ),
    mesh=_MESH,
    compiler_params=pltpu.CompilerParams(needs_layout_passes=False, use_tc_tiling_on_sc=False),
    scratch_types=[
        pltpu.VMEM((RPS,), _f32),
        pltpu.VMEM((GPC, 3, EPG), _i32),
        pltpu.VMEM((GPC, 3, EPG), _i32),
        pltpu.VMEM((CHUNK, H), _f32),
        pltpu.VMEM((CHUNK, H), _f32),
        pltpu.VMEM_SHARED((NP, H), _f32),
        pltpu.SemaphoreType.DMA,
        pltpu.SemaphoreType.DMA,
        pltpu.SemaphoreType.DMA,
        pltpu.SemaphoreType.DMA,
        pltpu.SemaphoreType.DMA,
        pltpu.SemaphoreType.DMA,
    ],
)(_sc_layer_body)


def _sc_hop1_body(m_hbm, row_hbm, col_hbm, w_hbm, out_hbm, rbuf, cbufv,
                  wbuf, mloc, accl):
    c = lax.axis_index("c")
    s = lax.axis_index("s")
    wid = c * NSUB + s
    pltpu.sync_copy(m_hbm, mloc)
    _zero_local(accl)

    def chunk(ci, carry):
        base = wid * 25600 + ci * CH1
        pltpu.sync_copy(row_hbm.at[pl.ds(base, CH1)], rbuf)
        pltpu.sync_copy(col_hbm.at[pl.ds(base, CH1)], cbufv)
        pltpu.sync_copy(w_hbm.at[pl.ds(base, CH1)], wbuf)

        def blk(i, cc):
            rv = rbuf[pl.ds(i * 16, 16)]
            cv = cbufv[pl.ds(i * 16, 16)]
            wv = wbuf[pl.ds(i * 16, 16)]
            mv = plsc.load_gather(mloc, [rv])
            plsc.addupdate_scatter(accl, [cv], mv * wv)
            return cc

        lax.fori_loop(0, CH1 // 16, blk, jnp.int32(0))
        return carry

    lax.fori_loop(0, 25600 // CH1, chunk, jnp.int32(0))
    _w1_writeback(accl, out_hbm, wid)


_sc_hop1 = functools.partial(
    pl.kernel,
    out_type=jax.ShapeDtypeStruct((32, NP), _f32),
    mesh=_MESH,
    compiler_params=pltpu.CompilerParams(needs_layout_passes=False, use_tc_tiling_on_sc=False),
    scratch_types=[
        pltpu.VMEM((CH1,), _i32),
        pltpu.VMEM((CH1,), _i32),
        pltpu.VMEM((CH1,), _f32),
        pltpu.VMEM((NP,), _f32),
        pltpu.VMEM((NP,), _f32),
    ],
)(_sc_hop1_body)


def _t0_body(degp_ref, x_ref, w_ref, dis_ref, dis2_ref, ma_ref, mb_ref, oa_ref):
    deg = jnp.sum(degp_ref[...], axis=0)
    mask = deg > 0
    dis = jnp.where(mask, lax.rsqrt(deg), 0.0)
    dis2 = jnp.where(mask, 1.0 / deg, 0.0)
    dis_ref[...] = dis
    dis2_ref[...] = dis2
    x = x_ref[...]
    m0 = x * dis[:, None]
    ma_ref[...] = m0[:, :H]
    mb_ref[...] = m0[:, H:]
    oa_ref[...] = jnp.dot(x, w_ref[...], preferred_element_type=_f32)


def _t0(degp, x, w10):
    return pl.pallas_call(
        _t0_body,
        grid=(NB,),
        in_specs=[
            pl.BlockSpec((32, BN), lambda i: (0, i)),
            pl.BlockSpec((BN, F), lambda i: (i, 0)),
            pl.BlockSpec((F, F), lambda i: (0, 0)),
        ],
        out_specs=[
            pl.BlockSpec((BN,), lambda i: (i,)),
            pl.BlockSpec((BN,), lambda i: (i,)),
            pl.BlockSpec((BN, H), lambda i: (i, 0)),
            pl.BlockSpec((BN, H), lambda i: (i, 0)),
            pl.BlockSpec((BN, F), lambda i: (i, 0)),
        ],
        out_shape=[
            jax.ShapeDtypeStruct((NP,), _f32),
            jax.ShapeDtypeStruct((NP,), _f32),
            jax.ShapeDtypeStruct((NP, H), _f32),
            jax.ShapeDtypeStruct((NP, H), _f32),
            jax.ShapeDtypeStruct((NP, F), _f32),
        ],
    )(degp, x, w10)


def _thop_body(ta_ref, tb_ref, dis_ref, oa_ref, w_ref, oao_ref):
    t = jnp.concatenate([ta_ref[...], tb_ref[...]], axis=1)
    td = t * dis_ref[...][:, None]
    oao_ref[...] = oa_ref[...] + jnp.dot(
        td, w_ref[...], preferred_element_type=_f32
    )


def _thop(ta, tb, dis, oa, wk):
    return pl.pallas_call(
        _thop_body,
        grid=(NB,),
        in_specs=[
            pl.BlockSpec((BN, H), lambda i: (i, 0)),
            pl.BlockSpec((BN, H), lambda i: (i, 0)),
            pl.BlockSpec((BN,), lambda i: (i,)),
            pl.BlockSpec((BN, F), lambda i: (i, 0)),
            pl.BlockSpec((F, F), lambda i: (0, 0)),
        ],
        out_specs=pl.BlockSpec((BN, F), lambda i: (i, 0)),
        out_shape=jax.ShapeDtypeStruct((NP, F), _f32),
    )(ta, tb, dis, oa, wk)


def _tlend_body(ta_ref, tb_ref, dis_ref, oa_ref, w_ref, b_ref, wn_ref,
                oao_ref, ma_ref, mb_ref):
    t = jnp.concatenate([ta_ref[...], tb_ref[...]], axis=1)
    dis = dis_ref[...]
    td = t * dis[:, None]
    h = oa_ref[...] + jnp.dot(td, w_ref[...], preferred_element_type=_f32)
    h = jnp.maximum(h + b_ref[...][None, :], 0.0)
    oao_ref[...] = jnp.dot(h, wn_ref[...], preferred_element_type=_f32)
    m = h * dis[:, None]
    ma_ref[...] = m[:, :H]
    mb_ref[...] = m[:, H:]


def _tlend(ta, tb, dis, oa, wk, b, wn0):
    return pl.pallas_call(
        _tlend_body,
        grid=(NB,),
        in_specs=[
            pl.BlockSpec((BN, H), lambda i: (i, 0)),
            pl.BlockSpec((BN, H), lambda i: (i, 0)),
            pl.BlockSpec((BN,), lambda i: (i,)),
            pl.BlockSpec((BN, F), lambda i: (i, 0)),
            pl.BlockSpec((F, F), lambda i: (0, 0)),
            pl.BlockSpec((F,), lambda i: (0,)),
            pl.BlockSpec((F, F), lambda i: (0, 0)),
        ],
        out_specs=[
            pl.BlockSpec((BN, F), lambda i: (i, 0)),
            pl.BlockSpec((BN, H), lambda i: (i, 0)),
            pl.BlockSpec((BN, H), lambda i: (i, 0)),
        ],
        out_shape=[
            jax.ShapeDtypeStruct((NP, F), _f32),
            jax.ShapeDtypeStruct((NP, H), _f32),
            jax.ShapeDtypeStruct((NP, H), _f32),
        ],
    )(ta, tb, dis, oa, wk, b, wn0)


def _tl2end_body(ta_ref, tb_ref, dis_ref, oa_ref, w_ref, b_ref, w3_ref,
                 v_ref, m3_ref):
    t = jnp.concatenate([ta_ref[...], tb_ref[...]], axis=1)
    dis = dis_ref[...]
    td = t * dis[:, None]
    h = oa_ref[...] + jnp.dot(td, w_ref[...], preferred_element_type=_f32)
    h = jnp.maximum(h + b_ref[...][None, :], 0.0)
    v = jnp.dot(h, w3_ref[...], preferred_element_type=_f32)
    v_ref[...] = v
    m3_ref[...] = dis * v[:, 3]


def _tl2end(ta, tb, dis, oa, wk, b, w3c):
    return pl.pallas_call(
        _tl2end_body,
        grid=(NB,),
        in_specs=[
            pl.BlockSpec((BN, H), lambda i: (i, 0)),
            pl.BlockSpec((BN, H), lambda i: (i, 0)),
            pl.BlockSpec((BN,), lambda i: (i,)),
            pl.BlockSpec((BN, F), lambda i: (i, 0)),
            pl.BlockSpec((F, F), lambda i: (0, 0)),
            pl.BlockSpec((F,), lambda i: (0,)),
            pl.BlockSpec((F, 4), lambda i: (0, 0)),
        ],
        out_specs=[
            pl.BlockSpec((BN, 4), lambda i: (i, 0)),
            pl.BlockSpec((BN,), lambda i: (i,)),
        ],
        out_shape=[
            jax.ShapeDtypeStruct((NP, 4), _f32),
            jax.ShapeDtypeStruct((NP,), _f32),
        ],
    )(ta, tb, dis, oa, wk, b, w3c)


def _tw1_body(p_ref, v_ref, dis_ref, dis2_ref, m_ref, *, k):
    t = jnp.sum(p_ref[...], axis=0)
    m_ref[...] = dis_ref[...] * v_ref[:, k] + dis2_ref[...] * t


def _tw1(p, v, dis, dis2, k):
    return pl.pallas_call(
        functools.partial(_tw1_body, k=k),
        grid=(NB,),
        in_specs=[
            pl.BlockSpec((32, BN), lambda i: (0, i)),
            pl.BlockSpec((BN, 4), lambda i: (i, 0)),
            pl.BlockSpec((BN,), lambda i: (i,)),
            pl.BlockSpec((BN,), lambda i: (i,)),
        ],
        out_specs=pl.BlockSpec((BN,), lambda i: (i,)),
        out_shape=jax.ShapeDtypeStruct((NP,), _f32),
    )(p, v, dis, dis2)


def _tfinal_body(p_ref, v_ref, dis_ref, b3_ref, batch_ref, y_ref):
    i = pl.program_id(0)

    @pl.when(i == 0)
    def _():
        y_ref[...] = jnp.zeros_like(y_ref)

    out3 = v_ref[:, 0] + dis_ref[...] * jnp.sum(p_ref[...], axis=0) + b3_ref[0]
    b = batch_ref[0]
    onehot = (
        b[None, :] == lax.broadcasted_iota(jnp.int32, (G, 1), 0)
    ).astype(_f32)
    y_ref[...] += onehot @ out3[:, None]

    @pl.when(i == NB - 1)
    def _():
        y_ref[...] = jax.nn.sigmoid(y_ref[...])


def _tfinal(p, v, dis, b3, batch2d):
    return pl.pallas_call(
        _tfinal_body,
        grid=(NB,),
        in_specs=[
            pl.BlockSpec((32, BN), lambda i: (0, i)),
            pl.BlockSpec((BN, 4), lambda i: (i, 0)),
            pl.BlockSpec((BN,), lambda i: (i,)),
            pl.BlockSpec((1,), lambda i: (0,)),
            pl.BlockSpec((1, BN), lambda i: (0, i)),
        ],
        out_specs=pl.BlockSpec((G, 1), lambda i: (0, 0)),
        out_shape=jax.ShapeDtypeStruct((G, 1), _f32),
    )(p, v, dis, b3, batch2d)


def _tlayer1_body(t1a, t1b, t2a, t2b, t3a, t3b, dis_ref, oa_ref, w1_ref,
                  b_ref, wn_ref, oao_ref, ma_ref, mb_ref):
    dis = dis_ref[...]
    acc = oa_ref[...]
    for k, (ta, tb) in enumerate(((t1a, t1b), (t2a, t2b), (t3a, t3b))):
        t = jnp.concatenate([ta[...], tb[...]], axis=1)
        acc = acc + jnp.dot(t * dis[:, None], w1_ref[k + 1],
                            preferred_element_type=_f32)
    h = jnp.maximum(acc + b_ref[...][None, :], 0.0)
    oao_ref[...] = jnp.dot(h, wn_ref[...], preferred_element_type=_f32)
    m = h * dis[:, None]
    ma_ref[...] = m[:, :H]
    mb_ref[...] = m[:, H:]


def _tlayer1(t1a, t1b, t2a, t2b, t3a, t3b, dis, oa, w1, b, wn0):
    tspec = pl.BlockSpec((BN, H), lambda i: (i, 0))
    return pl.pallas_call(
        _tlayer1_body,
        grid=(NB,),
        in_specs=[
            tspec, tspec, tspec, tspec, tspec, tspec,
            pl.BlockSpec((BN,), lambda i: (i,)),
            pl.BlockSpec((BN, F), lambda i: (i, 0)),
            pl.BlockSpec((4, F, F), lambda i: (0, 0, 0)),
            pl.BlockSpec((F,), lambda i: (0,)),
            pl.BlockSpec((F, F), lambda i: (0, 0)),
        ],
        out_specs=[
            pl.BlockSpec((BN, F), lambda i: (i, 0)),
            pl.BlockSpec((BN, H), lambda i: (i, 0)),
            pl.BlockSpec((BN, H), lambda i: (i, 0)),
        ],
        out_shape=[
            jax.ShapeDtypeStruct((NP, F), _f32),
            jax.ShapeDtypeStruct((NP, H), _f32),
            jax.ShapeDtypeStruct((NP, H), _f32),
        ],
    )(t1a, t1b, t2a, t2b, t3a, t3b, dis, oa, w1, b, wn0)


def _tlayer2_body(t1a, t1b, t2a, t2b, t3a, t3b, dis_ref, oa_ref, w2_ref,
                  b_ref, w3_ref, v_ref, m3_ref):
    dis = dis_ref[...]
    acc = oa_ref[...]
    for k, (ta, tb) in enumerate(((t1a, t1b), (t2a, t2b), (t3a, t3b))):
        t = jnp.concatenate([ta[...], tb[...]], axis=1)
        acc = acc + jnp.dot(t * dis[:, None], w2_ref[k + 1],
                            preferred_element_type=_f32)
    h = jnp.maximum(acc + b_ref[...][None, :], 0.0)
    v = jnp.dot(h, w3_ref[...], preferred_element_type=_f32)
    v_ref[...] = v
    m3_ref[...] = dis * v[:, 3]


def _tlayer2(t1a, t1b, t2a, t2b, t3a, t3b, dis, oa, w2, b, w3c):
    tspec = pl.BlockSpec((BN, H), lambda i: (i, 0))
    return pl.pallas_call(
        _tlayer2_body,
        grid=(NB,),
        in_specs=[
            tspec, tspec, tspec, tspec, tspec, tspec,
            pl.BlockSpec((BN,), lambda i: (i,)),
            pl.BlockSpec((BN, F), lambda i: (i, 0)),
            pl.BlockSpec((4, F, F), lambda i: (0, 0, 0)),
            pl.BlockSpec((F,), lambda i: (0,)),
            pl.BlockSpec((F, 4), lambda i: (0, 0)),
        ],
        out_specs=[
            pl.BlockSpec((BN, 4), lambda i: (i, 0)),
            pl.BlockSpec((BN,), lambda i: (i,)),
        ],
        out_shape=[
            jax.ShapeDtypeStruct((NP, 4), _f32),
            jax.ShapeDtypeStruct((NP,), _f32),
        ],
    )(t1a, t1b, t2a, t2b, t3a, t3b, dis, oa, w2, b, w3c)


def kernel(x, batch, edge_index, edge_weight, W1, b1, W2, b2, W3, b3):
    row = edge_index[0]
    col = edge_index[1]
    rowp = jnp.concatenate([row, jnp.zeros((EP - E,), _i32)])
    colp = jnp.concatenate([col, jnp.zeros((EP - E,), _i32)])
    wp = jnp.concatenate([edge_weight, jnp.zeros((EP - E,), _f32)])
    col2d = colp.reshape(EP // EPG, EPG)
    epack = jnp.stack(
        [rowp.reshape(EP // EPG, EPG), col2d,
         wp.view(_i32).reshape(EP // EPG, EPG)], axis=1,
    )
    x_pad = jnp.zeros((NP, F), _f32).at[:N].set(x)
    batch2d = jnp.full((NP,), -1, _i32).at[:N].set(batch).reshape(1, NP)
    z2 = jnp.zeros((RPS, H), _f32)
    w3c = jnp.transpose(W3[:, :, 0])

    degp = _sc_deg(colp, wp)
    dis, dis2, ma, mb, oa = _t0(degp, x_pad, W1[0])

    t1a, t1b, t2a, t2b, t3a, t3b = _sc_layer(ma, mb, epack, z2, dis2)[:6]
    oa, ma, mb = _tlayer1(t1a, t1b, t2a, t2b, t3a, t3b, dis, oa, W1, b1,
                          W2[0])
    t1a, t1b, t2a, t2b, t3a, t3b = _sc_layer(ma, mb, epack, z2, dis2)[:6]
    v, m = _tlayer2(t1a, t1b, t2a, t2b, t3a, t3b, dis, oa, W2, b2, w3c)

    for k in (2, 1):
        p = _sc_hop1(m, rowp, colp, wp)
        m = _tw1(p, v, dis, dis2, k)
    p = _sc_hop1(m, rowp, colp, wp)
    return _tfinal(p, v, dis, b3, batch2d)


# speedup vs baseline: 1.1408x; 1.0046x over previous
import functools

import jax
import jax.numpy as jnp
from jax import lax
from jax.experimental import pallas as pl
from jax.experimental.pallas import tpu as pltpu
from jax.experimental.pallas import tpu_sc as plsc

N = 50000
E = 800000
G = 32
F = 64
H = 32

NP = 51200
EP = 819200
EPG = 128
GPC = 2
CHUNK = EPG * GPC
NSUB = 16
RPS = NP // NSUB
BN = 2048
NB = NP // BN

_MESH = plsc.VectorSubcoreMesh(core_axis_name="c", subcore_axis_name="s")
_f32 = jnp.float32
_i32 = jnp.int32


def _zero_slice(zsrc, acc, s):
    pltpu.sync_copy(zsrc, acc.at[pl.ds(s * RPS, RPS)])


CH1 = 3200


def _zero_local(accl):
    def zb(i, cc):
        accl[pl.ds(i * 16, 16)] = jnp.zeros((16,), _f32)
        return cc
    lax.fori_loop(0, NP // 16, zb, jnp.int32(0))


def _w1_writeback(accl, out_hbm, wid):
    pltpu.sync_copy(accl, out_hbm.at[wid])


def _sc_deg_body(col_hbm, w_hbm, out_hbm, cbufv, wbuf, accl):
    c = lax.axis_index("c")
    s = lax.axis_index("s")
    wid = c * NSUB + s
    _zero_local(accl)

    def chunk(ci, carry):
        base = wid * 25600 + ci * CH1
        pltpu.sync_copy(col_hbm.at[pl.ds(base, CH1)], cbufv)
        pltpu.sync_copy(w_hbm.at[pl.ds(base, CH1)], wbuf)

        def blk(i, cc):
            cv = cbufv[pl.ds(i * 16, 16)]
            wv = wbuf[pl.ds(i * 16, 16)]
            plsc.addupdate_scatter(accl, [cv], wv)
            return cc

        lax.fori_loop(0, CH1 // 16, blk, jnp.int32(0))
        return carry

    lax.fori_loop(0, 25600 // CH1, chunk, jnp.int32(0))
    _w1_writeback(accl, out_hbm, wid)


_sc_deg = functools.partial(
    pl.kernel,
    out_type=jax.ShapeDtypeStruct((32, NP), _f32),
    mesh=_MESH,
    compiler_params=pltpu.CompilerParams(needs_layout_passes=False, use_tc_tiling_on_sc=False),
    scratch_types=[
        pltpu.VMEM((CH1,), _i32),
        pltpu.VMEM((CH1,), _f32),
        pltpu.VMEM((NP,), _f32),
    ],
)(_sc_deg_body)


def _hop32_gather(m_hbm, ebuf, gbuf, sem, gbase):
    return [
        pltpu.async_copy(
            m_hbm.at[ebuf.at[g, 0]], gbuf.at[pl.ds(g * EPG, EPG)], sem
        )
        for g in range(GPC)
    ]


def _hop32_drain_gather(m_hbm, ebuf, gbuf, sem):
    for g in range(GPC):
        pltpu.make_async_copy(
            m_hbm.at[ebuf.at[g, 0]], gbuf.at[pl.ds(g * EPG, EPG)], sem
        ).wait()


def _hop32_scale(ebuf, gbuf):
    for g in range(GPC):
        def blk(i, cc, g=g):
            wv = plsc.bitcast(ebuf[g, 2, pl.ds(i * 16, 16)], _f32)
            for jj in range(16):
                e = g * EPG + i * 16 + jj
                sp = lax.gather(
                    wv, jnp.full((16, 1), jj, _i32),
                    lax.GatherDimensionNumbers(
                        offset_dims=(), collapsed_slice_dims=(0,),
                        start_index_map=(0,)),
                    (1,), mode=lax.GatherScatterMode.PROMISE_IN_BOUNDS)
                gbuf[e, pl.ds(0, 16)] = gbuf[e, pl.ds(0, 16)] * sp
                gbuf[e, pl.ds(16, 16)] = gbuf[e, pl.ds(16, 16)] * sp
            return cc
        lax.fori_loop(0, EPG // 16, blk, jnp.int32(0))


def _hop32_scatter(acc, ebuf, gbuf, sem):
    return [
        pltpu.async_copy(
            gbuf.at[pl.ds(g * EPG, EPG)], acc.at[ebuf.at[g, 1]], sem, add=True
        )
        for g in range(GPC)
    ]


def _hop32_half(m_hbm, t_hbm, mo_hbm, s, e_hbm, z2_hbm, dbuf, ebuf0, ebuf1,
                gbuf0, gbuf1, acc, gsem0, gsem1, ssem0, ssem1, isem0, isem1):
    nch = 400 // GPC
    gps = s * 400

    pltpu.sync_copy(e_hbm.at[pl.ds(gps, GPC)], ebuf0)
    pltpu.sync_copy(e_hbm.at[pl.ds(gps + GPC, GPC)], ebuf1)
    _hop32_gather(m_hbm, ebuf0, gbuf0, gsem0, 0)
    _hop32_gather(m_hbm, ebuf1, gbuf1, gsem1, 0)

    def body(i, carry):
        p0 = jnp.minimum(2 * i + 2, nch - 1)
        p1 = jnp.minimum(2 * i + 3, nch - 1)
        _hop32_drain_gather(m_hbm, ebuf0, gbuf0, gsem0)
        _hop32_scale(ebuf0, gbuf0)
        s0 = _hop32_scatter(acc, ebuf0, gbuf0, ssem0)
        _hop32_drain_gather(m_hbm, ebuf1, gbuf1, gsem1)
        _hop32_scale(ebuf1, gbuf1)
        s1 = _hop32_scatter(acc, ebuf1, gbuf1, ssem1)
        for d in s0:
            d.wait()
        i0 = pltpu.async_copy(e_hbm.at[pl.ds(gps + p0 * GPC, GPC)], ebuf0,
                              isem0)
        for d in s1:
            d.wait()
        i1 = pltpu.async_copy(e_hbm.at[pl.ds(gps + p1 * GPC, GPC)], ebuf1,
                              isem1)
        i0.wait()
        _hop32_gather(m_hbm, ebuf0, gbuf0, gsem0, 0)
        i1.wait()
        _hop32_gather(m_hbm, ebuf1, gbuf1, gsem1, 0)
        return carry

    lax.fori_loop(0, nch // 2, body, jnp.int32(0))
    _hop32_drain_gather(m_hbm, ebuf0, gbuf0, gsem0)
    _hop32_drain_gather(m_hbm, ebuf1, gbuf1, gsem1)
    plsc.subcore_barrier()
    wb = pltpu.async_copy(acc.at[pl.ds(s * RPS, RPS)],
                          t_hbm.at[pl.ds(s * RPS, RPS)], gsem0)
    if mo_hbm is not None:
        base = s * RPS
        off = 0
        for rows in [256] * 12 + [128]:
            pltpu.sync_copy(acc.at[pl.ds(base + off, rows)],
                            gbuf0.at[pl.ds(0, rows)])

            def rblk(i, cc, off=off):
                dv = dbuf[pl.ds(off + i * 16, 16)]
                for jj in range(16):
                    r = i * 16 + jj
                    sp = lax.gather(
                        dv, jnp.full((16, 1), jj, _i32),
                        lax.GatherDimensionNumbers(
                            offset_dims=(), collapsed_slice_dims=(0,),
                            start_index_map=(0,)),
                        (1,), mode=lax.GatherScatterMode.PROMISE_IN_BOUNDS)
                    gbuf0[r, pl.ds(0, 16)] = gbuf0[r, pl.ds(0, 16)] * sp
                    gbuf0[r, pl.ds(16, 16)] = gbuf0[r, pl.ds(16, 16)] * sp
                return cc

            lax.fori_loop(0, rows // 16, rblk, jnp.int32(0))
            pltpu.sync_copy(gbuf0.at[pl.ds(0, rows)],
                            mo_hbm.at[pl.ds(base + off, rows)])
            off += rows
    wb.wait()
    pltpu.sync_copy(z2_hbm, acc.at[pl.ds(s * RPS, RPS)])
    plsc.subcore_barrier()


def _sc_layer_body(ma_hbm, mb_hbm, e_hbm, z2_hbm, dis2_hbm, t1a, t1b, t2a,
                   t2b, t3a, t3b, m1a, m1b, m2a, m2b, dbuf, ebuf0, ebuf1,
                   gbuf0, gbuf1, acc, gsem0, gsem1, ssem0, ssem1, isem0,
                   isem1):
    c = lax.axis_index("c")
    s = lax.axis_index("s")
    pltpu.sync_copy(dis2_hbm.at[pl.ds(s * RPS, RPS)], dbuf)
    pltpu.sync_copy(z2_hbm, acc.at[pl.ds(s * RPS, RPS)])
    plsc.subcore_barrier()

    @pl.when(c == 0)
    def _():
        for m_in, t_out, m_out in ((ma_hbm, t1a, m1a), (m1a, t2a, m2a),
                                   (m2a, t3a, None)):
            _hop32_half(m_in, t_out, m_out, s, e_hbm, z2_hbm, dbuf, ebuf0,
                        ebuf1, gbuf0, gbuf1, acc, gsem0, gsem1, ssem0, ssem1,
                        isem0, isem1)

    @pl.when(c == 1)
    def _():
        for m_in, t_out, m_out in ((mb_hbm, t1b, m1b), (m1b, t2b, m2b),
                                   (m2b, t3b, None)):
            _hop32_half(m_in, t_out, m_out, s, e_hbm, z2_hbm, dbuf, ebuf0,
                        ebuf1, gbuf0, gbuf1, acc, gsem0, gsem1, ssem0, ssem1,
                        isem0, isem1)


_sc_layer = functools.partial(
    pl.kernel,
    out_type=tuple(
        jax.ShapeDtypeStruct((NP, H), _f32) for _ in range(10)
    ),
    mesh=_MESH,
    compiler_params=pltpu.CompilerParams(needs_layout_passes=False, use_tc_tiling_on_sc=False),
    scratch_types=[
        pltpu.VMEM((RPS,), _f32),
        pltpu.VMEM((GPC, 3, EPG), _i32),
        pltpu.VMEM((GPC, 3, EPG), _i32),
        pltpu.VMEM((CHUNK, H), _f32),
        pltpu.VMEM((CHUNK, H), _f32),
        pltpu.VMEM_SHARED((NP, H), _f32),
        pltpu.SemaphoreType.DMA,
        pltpu.SemaphoreType.DMA,
        pltpu.SemaphoreType.DMA,
        pltpu.SemaphoreType.DMA,
        pltpu.SemaphoreType.DMA,
        pltpu.SemaphoreType.DMA,
    ],
)(_sc_layer_body)


def _sc_hop1_body(m_hbm, row_hbm, col_hbm, w_hbm, out_hbm, rbuf, cbufv,
                  wbuf, mloc, accl):
    c = lax.axis_index("c")
    s = lax.axis_index("s")
    wid = c * NSUB + s
    pltpu.sync_copy(m_hbm, mloc)
    _zero_local(accl)

    def chunk(ci, carry):
        base = wid * 25600 + ci * CH1
        pltpu.sync_copy(row_hbm.at[pl.ds(base, CH1)], rbuf)
        pltpu.sync_copy(col_hbm.at[pl.ds(base, CH1)], cbufv)
        pltpu.sync_copy(w_hbm.at[pl.ds(base, CH1)], wbuf)

        def blk(i, cc):
            rv = rbuf[pl.ds(i * 16, 16)]
            cv = cbufv[pl.ds(i * 16, 16)]
            wv = wbuf[pl.ds(i * 16, 16)]
            mv = plsc.load_gather(mloc, [rv])
            plsc.addupdate_scatter(accl, [cv], mv * wv)
            return cc

        lax.fori_loop(0, CH1 // 16, blk, jnp.int32(0))
        return carry

    lax.fori_loop(0, 25600 // CH1, chunk, jnp.int32(0))
    _w1_writeback(accl, out_hbm, wid)


_sc_hop1 = functools.partial(
    pl.kernel,
    out_type=jax.ShapeDtypeStruct((32, NP), _f32),
    mesh=_MESH,
    compiler_params=pltpu.CompilerParams(needs_layout_passes=False, use_tc_tiling_on_sc=False),
    scratch_types=[
        pltpu.VMEM((CH1,), _i32),
        pltpu.VMEM((CH1,), _i32),
        pltpu.VMEM((CH1,), _f32),
        pltpu.VMEM((NP,), _f32),
        pltpu.VMEM((NP,), _f32),
    ],
)(_sc_hop1_body)


def _t0_body(degp_ref, x_ref, w_ref, dis_ref, dis2_ref, ma_ref, mb_ref, oa_ref):
    deg = jnp.sum(degp_ref[...], axis=0)
    mask = deg > 0
    dis = jnp.where(mask, lax.rsqrt(deg), 0.0)
    dis2 = jnp.where(mask, 1.0 / deg, 0.0)
    dis_ref[...] = dis
    dis2_ref[...] = dis2
    x = x_ref[...]
    m0 = x * dis[:, None]
    ma_ref[...] = m0[:, :H]
    mb_ref[...] = m0[:, H:]
    oa_ref[...] = jnp.dot(x, w_ref[...], preferred_element_type=_f32)


def _t0(degp, x, w10):
    return pl.pallas_call(
        _t0_body,
        grid=(NB,),
        in_specs=[
            pl.BlockSpec((32, BN), lambda i: (0, i)),
            pl.BlockSpec((BN, F), lambda i: (i, 0)),
            pl.BlockSpec((F, F), lambda i: (0, 0)),
        ],
        out_specs=[
            pl.BlockSpec((BN,), lambda i: (i,)),
            pl.BlockSpec((BN,), lambda i: (i,)),
            pl.BlockSpec((BN, H), lambda i: (i, 0)),
            pl.BlockSpec((BN, H), lambda i: (i, 0)),
            pl.BlockSpec((BN, F), lambda i: (i, 0)),
        ],
        out_shape=[
            jax.ShapeDtypeStruct((NP,), _f32),
            jax.ShapeDtypeStruct((NP,), _f32),
            jax.ShapeDtypeStruct((NP, H), _f32),
            jax.ShapeDtypeStruct((NP, H), _f32),
            jax.ShapeDtypeStruct((NP, F), _f32),
        ],
    )(degp, x, w10)


def _thop_body(ta_ref, tb_ref, dis_ref, oa_ref, w_ref, oao_ref):
    t = jnp.concatenate([ta_ref[...], tb_ref[...]], axis=1)
    td = t * dis_ref[...][:, None]
    oao_ref[...] = oa_ref[...] + jnp.dot(
        td, w_ref[...], preferred_element_type=_f32
    )


def _thop(ta, tb, dis, oa, wk):
    return pl.pallas_call(
        _thop_body,
        grid=(NB,),
        in_specs=[
            pl.BlockSpec((BN, H), lambda i: (i, 0)),
            pl.BlockSpec((BN, H), lambda i: (i, 0)),
            pl.BlockSpec((BN,), lambda i: (i,)),
            pl.BlockSpec((BN, F), lambda i: (i, 0)),
            pl.BlockSpec((F, F), lambda i: (0, 0)),
        ],
        out_specs=pl.BlockSpec((BN, F), lambda i: (i, 0)),
        out_shape=jax.ShapeDtypeStruct((NP, F), _f32),
    )(ta, tb, dis, oa, wk)


def _tlend_body(ta_ref, tb_ref, dis_ref, oa_ref, w_ref, b_ref, wn_ref,
                oao_ref, ma_ref, mb_ref):
    t = jnp.concatenate([ta_ref[...], tb_ref[...]], axis=1)
    dis = dis_ref[...]
    td = t * dis[:, None]
    h = oa_ref[...] + jnp.dot(td, w_ref[...], preferred_element_type=_f32)
    h = jnp.maximum(h + b_ref[...][None, :], 0.0)
    oao_ref[...] = jnp.dot(h, wn_ref[...], preferred_element_type=_f32)
    m = h * dis[:, None]
    ma_ref[...] = m[:, :H]
    mb_ref[...] = m[:, H:]


def _tlend(ta, tb, dis, oa, wk, b, wn0):
    return pl.pallas_call(
        _tlend_body,
        grid=(NB,),
        in_specs=[
            pl.BlockSpec((BN, H), lambda i: (i, 0)),
            pl.BlockSpec((BN, H), lambda i: (i, 0)),
            pl.BlockSpec((BN,), lambda i: (i,)),
            pl.BlockSpec((BN, F), lambda i: (i, 0)),
            pl.BlockSpec((F, F), lambda i: (0, 0)),
            pl.BlockSpec((F,), lambda i: (0,)),
            pl.BlockSpec((F, F), lambda i: (0, 0)),
        ],
        out_specs=[
            pl.BlockSpec((BN, F), lambda i: (i, 0)),
            pl.BlockSpec((BN, H), lambda i: (i, 0)),
            pl.BlockSpec((BN, H), lambda i: (i, 0)),
        ],
        out_shape=[
            jax.ShapeDtypeStruct((NP, F), _f32),
            jax.ShapeDtypeStruct((NP, H), _f32),
            jax.ShapeDtypeStruct((NP, H), _f32),
        ],
    )(ta, tb, dis, oa, wk, b, wn0)


def _tl2end_body(ta_ref, tb_ref, dis_ref, oa_ref, w_ref, b_ref, w3_ref,
                 v_ref, m3_ref):
    t = jnp.concatenate([ta_ref[...], tb_ref[...]], axis=1)
    dis = dis_ref[...]
    td = t * dis[:, None]
    h = oa_ref[...] + jnp.dot(td, w_ref[...], preferred_element_type=_f32)
    h = jnp.maximum(h + b_ref[...][None, :], 0.0)
    v = jnp.dot(h, w3_ref[...], preferred_element_type=_f32)
    v_ref[...] = v
    m3_ref[...] = dis * v[:, 3]


def _tl2end(ta, tb, dis, oa, wk, b, w3c):
    return pl.pallas_call(
        _tl2end_body,
        grid=(NB,),
        in_specs=[
            pl.BlockSpec((BN, H), lambda i: (i, 0)),
            pl.BlockSpec((BN, H), lambda i: (i, 0)),
            pl.BlockSpec((BN,), lambda i: (i,)),
            pl.BlockSpec((BN, F), lambda i: (i, 0)),
            pl.BlockSpec((F, F), lambda i: (0, 0)),
            pl.BlockSpec((F,), lambda i: (0,)),
            pl.BlockSpec((F, 4), lambda i: (0, 0)),
        ],
        out_specs=[
            pl.BlockSpec((BN, 4), lambda i: (i, 0)),
            pl.BlockSpec((BN,), lambda i: (i,)),
        ],
        out_shape=[
            jax.ShapeDtypeStruct((NP, 4), _f32),
            jax.ShapeDtypeStruct((NP,), _f32),
        ],
    )(ta, tb, dis, oa, wk, b, w3c)


def _tw1_body(p_ref, v_ref, dis_ref, dis2_ref, m_ref, *, k):
    t = jnp.sum(p_ref[...], axis=0)
    m_ref[...] = dis_ref[...] * v_ref[:, k] + dis2_ref[...] * t


def _tw1(p, v, dis, dis2, k):
    return pl.pallas_call(
        functools.partial(_tw1_body, k=k),
        grid=(NB,),
        in_specs=[
            pl.BlockSpec((32, BN), lambda i: (0, i)),
            pl.BlockSpec((BN, 4), lambda i: (i, 0)),
            pl.BlockSpec((BN,), lambda i: (i,)),
            pl.BlockSpec((BN,), lambda i: (i,)),
        ],
        out_specs=pl.BlockSpec((BN,), lambda i: (i,)),
        out_shape=jax.ShapeDtypeStruct((NP,), _f32),
    )(p, v, dis, dis2)


def _tfinal_body(p_ref, v_ref, dis_ref, b3_ref, batch_ref, y_ref):
    i = pl.program_id(0)

    @pl.when(i == 0)
    def _():
        y_ref[...] = jnp.zeros_like(y_ref)

    out3 = v_ref[:, 0] + dis_ref[...] * jnp.sum(p_ref[...], axis=0) + b3_ref[0]
    b = batch_ref[0]
    onehot = (
        b[None, :] == lax.broadcasted_iota(jnp.int32, (G, 1), 0)
    ).astype(_f32)
    y_ref[...] += onehot @ out3[:, None]

    @pl.when(i == NB - 1)
    def _():
        y_ref[...] = jax.nn.sigmoid(y_ref[...])


def _tfinal(p, v, dis, b3, batch2d):
    return pl.pallas_call(
        _tfinal_body,
        grid=(NB,),
        in_specs=[
            pl.BlockSpec((32, BN), lambda i: (0, i)),
            pl.BlockSpec((BN, 4), lambda i: (i, 0)),
            pl.BlockSpec((BN,), lambda i: (i,)),
            pl.BlockSpec((1,), lambda i: (0,)),
            pl.BlockSpec((1, BN), lambda i: (0, i)),
        ],
        out_specs=pl.BlockSpec((G, 1), lambda i: (0, 0)),
        out_shape=jax.ShapeDtypeStruct((G, 1), _f32),
    )(p, v, dis, b3, batch2d)


def _tlayer1_body(t1a, t1b, t2a, t2b, t3a, t3b, dis_ref, oa_ref, w1_ref,
                  b_ref, wn_ref, oao_ref, ma_ref, mb_ref):
    dis = dis_ref[...]
    acc = oa_ref[...]
    for k, (ta, tb) in enumerate(((t1a, t1b), (t2a, t2b), (t3a, t3b))):
        t = jnp.concatenate([ta[...], tb[...]], axis=1)
        acc = acc + jnp.dot(t * dis[:, None], w1_ref[k + 1],
                            preferred_element_type=_f32)
    h = jnp.maximum(acc + b_ref[...][None, :], 0.0)
    oao_ref[...] = jnp.dot(h, wn_ref[...], preferred_element_type=_f32)
    m = h * dis[:, None]
    ma_ref[...] = m[:, :H]
    mb_ref[...] = m[:, H:]


def _tlayer1(t1a, t1b, t2a, t2b, t3a, t3b, dis, oa, w1, b, wn0):
    tspec = pl.BlockSpec((BN, H), lambda i: (i, 0))
    return pl.pallas_call(
        _tlayer1_body,
        grid=(NB,),
        in_specs=[
            tspec, tspec, tspec, tspec, tspec, tspec,
            pl.BlockSpec((BN,), lambda i: (i,)),
            pl.BlockSpec((BN, F), lambda i: (i, 0)),
            pl.BlockSpec((4, F, F), lambda i: (0, 0, 0)),
            pl.BlockSpec((F,), lambda i: (0,)),
            pl.BlockSpec((F, F), lambda i: (0, 0)),
        ],
        out_specs=[
            pl.BlockSpec((BN, F), lambda i: (i, 0)),
            pl.BlockSpec((BN, H), lambda i: (i, 0)),
            pl.BlockSpec((BN, H), lambda i: (i, 0)),
        ],
        out_shape=[
            jax.ShapeDtypeStruct((NP, F), _f32),
            jax.ShapeDtypeStruct((NP, H), _f32),
            jax.ShapeDtypeStruct((NP, H), _f32),
        ],
    )(t1a, t1b, t2a, t2b, t3a, t3b, dis, oa, w1, b, wn0)


def _tlayer2_body(t1a, t1b, t2a, t2b, t3a, t3b, dis_ref, oa_ref, w2_ref,
                  b_ref, w3_ref, v_ref, m3_ref):
    dis = dis_ref[...]
    acc = oa_ref[...]
    for k, (ta, tb) in enumerate(((t1a, t1b), (t2a, t2b), (t3a, t3b))):
        t = jnp.concatenate([ta[...], tb[...]], axis=1)
        acc = acc + jnp.dot(t * dis[:, None], w2_ref[k + 1],
                            preferred_element_type=_f32)
    h = jnp.maximum(acc + b_ref[...][None, :], 0.0)
    v = jnp.dot(h, w3_ref[...], preferred_element_type=_f32)
    v_ref[...] = v
    m3_ref[...] = dis * v[:, 3]


def _tlayer2(t1a, t1b, t2a, t2b, t3a, t3b, dis, oa, w2, b, w3c):
    tspec = pl.BlockSpec((BN, H), lambda i: (i, 0))
    return pl.pallas_call(
        _tlayer2_body,
        grid=(NB,),
        in_specs=[
            tspec, tspec, tspec, tspec, tspec, tspec,
            pl.BlockSpec((BN,), lambda i: (i,)),
            pl.BlockSpec((BN, F), lambda i: (i, 0)),
            pl.BlockSpec((4, F, F), lambda i: (0, 0, 0)),
            pl.BlockSpec((F,), lambda i: (0,)),
            pl.BlockSpec((F, 4), lambda i: (0, 0)),
        ],
        out_specs=[
            pl.BlockSpec((BN, 4), lambda i: (i, 0)),
            pl.BlockSpec((BN,), lambda i: (i,)),
        ],
        out_shape=[
            jax.ShapeDtypeStruct((NP, 4), _f32),
            jax.ShapeDtypeStruct((NP,), _f32),
        ],
    )(t1a, t1b, t2a, t2b, t3a, t3b, dis, oa, w2, b, w3c)


def kernel(x, batch, edge_index, edge_weight, W1, b1, W2, b2, W3, b3):
    row = edge_index[0]
    col = edge_index[1]
    rowp = jnp.concatenate([row, jnp.zeros((EP - E,), _i32)])
    colp = jnp.concatenate([col, jnp.zeros((EP - E,), _i32)])
    wp = jnp.concatenate([edge_weight, jnp.zeros((EP - E,), _f32)])
    col2d = colp.reshape(EP // EPG, EPG)
    epack = jnp.stack(
        [rowp.reshape(EP // EPG, EPG), col2d,
         wp.view(_i32).reshape(EP // EPG, EPG)], axis=1,
    )
    x_pad = jnp.zeros((NP, F), _f32).at[:N].set(x)
    batch2d = jnp.full((NP,), -1, _i32).at[:N].set(batch).reshape(1, NP)
    z2 = jnp.zeros((RPS, H), _f32)
    w3c = jnp.transpose(W3[:, :, 0])

    degp = _sc_deg(colp, wp)
    dis, dis2, ma, mb, oa = _t0(degp, x_pad, W1[0])

    t1a, t1b, t2a, t2b, t3a, t3b = _sc_layer(ma, mb, epack, z2, dis2)[:6]
    oa, ma, mb = _tlayer1(t1a, t1b, t2a, t2b, t3a, t3b, dis, oa, W1, b1,
                          W2[0])
    t1a, t1b, t2a, t2b, t3a, t3b = _sc_layer(ma, mb, epack, z2, dis2)[:6]
    v, m = _tlayer2(t1a, t1b, t2a, t2b, t3a, t3b, dis, oa, W2, b2, w3c)

    for k in (2, 1):
        p = _sc_hop1(m, rowp, colp, wp)
        m = _tw1(p, v, dis, dis2, k)
    p = _sc_hop1(m, rowp, colp, wp)
    return _tfinal(p, v, dis, b3, batch2d)
